# Initial kernel scaffold; baseline (speedup 1.0000x reference)
import jax
import jax.numpy as jnp
from jax.experimental import pallas as pl


def kernel(x, W1, b1, W2, b2, temperature):
    raise NotImplementedError("write your pallas kernel here")


# speedup vs baseline: 1.2776x; 1.2776x over previous
import functools

import jax
import jax.numpy as jnp
from jax.experimental import pallas as pl
from jax.experimental.pallas import tpu as pltpu

M_TILE = 256
H_TILE = 512
K = 8


def _router_body(x_ref, w1_ref, b1_ref, w2_ref, b2_ref, temp_ref,
                 mask_ref, loss_ref, weights_ref, idx_ref,
                 acc_ref, usage_ref, *, e, capacity):
    m = pl.program_id(0)
    h = pl.program_id(1)
    nm = pl.num_programs(0)
    nh = pl.num_programs(1)

    hid = jnp.dot(x_ref[...], w1_ref[...], preferred_element_type=jnp.float32)
    hid = hid + b1_ref[...]
    hid = hid * 0.5 * (1.0 + jax.lax.erf(hid * 0.7071067811865476))
    part = jnp.dot(hid, w2_ref[...], preferred_element_type=jnp.float32)

    @pl.when(h == 0)
    def _():
        acc_ref[...] = jnp.zeros_like(acc_ref)

    acc_ref[...] += part

    @pl.when(h == nh - 1)
    def _():
        scores = (acc_ref[...] + b2_ref[...]) / temp_ref[0, 0]
        lmax = jnp.max(scores, axis=-1, keepdims=True)
        ex = jnp.exp(scores - lmax)
        w = ex / jnp.sum(ex, axis=-1, keepdims=True)
        weights_ref[...] = w

        iota = jax.lax.broadcasted_iota(jnp.int32, w.shape, 1)
        rem = w
        sel = jnp.zeros(w.shape, jnp.bool_)
        vals, idxs = [], []
        for _ in range(K):
            mk = jnp.max(rem, axis=-1, keepdims=True)
            ik = jnp.min(jnp.where(rem == mk, iota, e), axis=-1, keepdims=True)
            hit = iota == ik
            sel = jnp.logical_or(sel, hit)
            rem = jnp.where(hit, -1.0, rem)
            vals.append(mk)
            idxs.append(ik)
        sum_k = vals[0]
        for v in vals[1:]:
            sum_k = sum_k + v
        mask = jnp.where(sel, w / sum_k, 0.0)
        mask_ref[...] = mask
        idx_ref[...] = jnp.concatenate(idxs, axis=-1)

        @pl.when(m == 0)
        def _():
            usage_ref[...] = jnp.zeros_like(usage_ref)

        usage_ref[...] += jnp.sum(mask, axis=0, keepdims=True)

        @pl.when(m == nm - 1)
        def _():
            usage = usage_ref[...]
            ideal = jnp.sum(usage) / e
            lb = jnp.mean((usage - ideal) ** 2)
            cl = jnp.mean(jnp.maximum(usage - capacity, 0.0))
            loss_ref[...] = jnp.full((1, 1), lb + cl, jnp.float32)


def kernel(x, W1, b1, W2, b2, temperature):
    B, S, D = x.shape
    Dh, E = W2.shape
    M = B * S
    xf = x.reshape(M, D)
    b1r = b1.reshape(1, Dh)
    b2r = b2.reshape(1, E)
    tr = temperature.reshape(1, 1)
    capacity = float(int(1.25 * S))

    nm = M // M_TILE
    nh = Dh // H_TILE

    body = functools.partial(_router_body, e=E, capacity=capacity)
    mask, loss, weights, idx = pl.pallas_call(
        body,
        grid=(nm, nh),
        in_specs=[
            pl.BlockSpec((M_TILE, D), lambda m, h: (m, 0)),
            pl.BlockSpec((D, H_TILE), lambda m, h: (0, h)),
            pl.BlockSpec((1, H_TILE), lambda m, h: (0, h)),
            pl.BlockSpec((H_TILE, E), lambda m, h: (h, 0)),
            pl.BlockSpec((1, E), lambda m, h: (0, 0)),
            pl.BlockSpec((1, 1), lambda m, h: (0, 0)),
        ],
        out_specs=[
            pl.BlockSpec((M_TILE, E), lambda m, h: (m, 0)),
            pl.BlockSpec((1, 1), lambda m, h: (0, 0)),
            pl.BlockSpec((M_TILE, E), lambda m, h: (m, 0)),
            pl.BlockSpec((M_TILE, K), lambda m, h: (m, 0)),
        ],
        out_shape=[
            jax.ShapeDtypeStruct((M, E), jnp.float32),
            jax.ShapeDtypeStruct((1, 1), jnp.float32),
            jax.ShapeDtypeStruct((M, E), jnp.float32),
            jax.ShapeDtypeStruct((M, K), jnp.int32),
        ],
        scratch_shapes=[
            pltpu.VMEM((M_TILE, E), jnp.float32),
            pltpu.VMEM((1, E), jnp.float32),
        ],
        compiler_params=pltpu.CompilerParams(
            dimension_semantics=("arbitrary", "arbitrary"),
        ),
    )(xf, W1, b1r, W2, b2r, tr)

    return (mask.reshape(B, S, E), loss.reshape(()),
            weights.reshape(B, S, E), idx.reshape(B, S, K))


# speedup vs baseline: 2.6991x; 2.1127x over previous
import functools

import jax
import jax.numpy as jnp
from jax.experimental import pallas as pl
from jax.experimental.pallas import tpu as pltpu

M_TILE = 512
K = 8


def _router_body(x_ref, w1_ref, b1_ref, w2_ref, b2_ref, temp_ref,
                 mask_ref, loss_ref, weights_ref, idx_ref,
                 usage_ref, *, e, capacity):
    m = pl.program_id(0)
    nm = pl.num_programs(0)

    hid = jnp.dot(x_ref[...], w1_ref[...], preferred_element_type=jnp.float32)
    hid = hid + b1_ref[...]
    hid = hid * 0.5 * (1.0 + jax.lax.erf(hid * 0.7071067811865476))
    scores = jnp.dot(hid, w2_ref[...], preferred_element_type=jnp.float32)
    scores = (scores + b2_ref[...]) / temp_ref[0, 0]

    lmax = jnp.max(scores, axis=-1, keepdims=True)
    ex = jnp.exp(scores - lmax)
    w = ex / jnp.sum(ex, axis=-1, keepdims=True)
    weights_ref[...] = w

    iota = jax.lax.broadcasted_iota(jnp.int32, w.shape, 1)
    rem = w
    sel = jnp.zeros(w.shape, jnp.bool_)
    vals, idxs = [], []
    for _ in range(K):
        mk = jnp.max(rem, axis=-1, keepdims=True)
        ik = jnp.min(jnp.where(rem == mk, iota, e), axis=-1, keepdims=True)
        hit = iota == ik
        sel = jnp.logical_or(sel, hit)
        rem = jnp.where(hit, -1.0, rem)
        vals.append(mk)
        idxs.append(ik)
    sum_k = vals[0]
    for v in vals[1:]:
        sum_k = sum_k + v
    mask = jnp.where(sel, w / sum_k, 0.0)
    mask_ref[...] = mask
    idx_ref[...] = jnp.concatenate(idxs, axis=-1)

    @pl.when(m == 0)
    def _():
        usage_ref[...] = jnp.zeros_like(usage_ref)

    usage_ref[...] += jnp.sum(mask, axis=0, keepdims=True)

    @pl.when(m == nm - 1)
    def _():
        usage = usage_ref[...]
        ideal = jnp.sum(usage) / e
        lb = jnp.mean((usage - ideal) ** 2)
        cl = jnp.mean(jnp.maximum(usage - capacity, 0.0))
        loss_ref[...] = jnp.full((1, 1), lb + cl, jnp.float32)


def kernel(x, W1, b1, W2, b2, temperature):
    B, S, D = x.shape
    Dh, E = W2.shape
    M = B * S
    xf = x.reshape(M, D)
    b1r = b1.reshape(1, Dh)
    b2r = b2.reshape(1, E)
    tr = temperature.reshape(1, 1)
    capacity = float(int(1.25 * S))

    nm = M // M_TILE

    body = functools.partial(_router_body, e=E, capacity=capacity)
    mask, loss, weights, idx = pl.pallas_call(
        body,
        grid=(nm,),
        in_specs=[
            pl.BlockSpec((M_TILE, D), lambda m: (m, 0)),
            pl.BlockSpec((D, Dh), lambda m: (0, 0)),
            pl.BlockSpec((1, Dh), lambda m: (0, 0)),
            pl.BlockSpec((Dh, E), lambda m: (0, 0)),
            pl.BlockSpec((1, E), lambda m: (0, 0)),
            pl.BlockSpec((1, 1), lambda m: (0, 0)),
        ],
        out_specs=[
            pl.BlockSpec((M_TILE, E), lambda m: (m, 0)),
            pl.BlockSpec((1, 1), lambda m: (0, 0)),
            pl.BlockSpec((M_TILE, E), lambda m: (m, 0)),
            pl.BlockSpec((M_TILE, K), lambda m: (m, 0)),
        ],
        out_shape=[
            jax.ShapeDtypeStruct((M, E), jnp.float32),
            jax.ShapeDtypeStruct((1, 1), jnp.float32),
            jax.ShapeDtypeStruct((M, E), jnp.float32),
            jax.ShapeDtypeStruct((M, K), jnp.int32),
        ],
        scratch_shapes=[
            pltpu.VMEM((1, E), jnp.float32),
        ],
        compiler_params=pltpu.CompilerParams(
            dimension_semantics=("arbitrary",),
        ),
    )(xf, W1, b1r, W2, b2r, tr)

    return (mask.reshape(B, S, E), loss.reshape(()),
            weights.reshape(B, S, E), idx.reshape(B, S, K))


# speedup vs baseline: 2.9036x; 1.0758x over previous
import functools

import jax
import jax.numpy as jnp
from jax.experimental import pallas as pl
from jax.experimental.pallas import tpu as pltpu

M_TILE = 1024
K = 8


def _router_body(x_ref, w1_ref, b1_ref, w2_ref, b2_ref, temp_ref,
                 mask_ref, loss_ref, weights_ref, idx_ref,
                 usage_ref, *, e, capacity):
    m = pl.program_id(0)
    nm = pl.num_programs(0)

    hid = jnp.dot(x_ref[...], w1_ref[...], preferred_element_type=jnp.float32)
    hid = hid + b1_ref[...]
    hid = hid * 0.5 * (1.0 + jax.lax.erf(hid * 0.7071067811865476))
    scores = jnp.dot(hid, w2_ref[...], preferred_element_type=jnp.float32)
    scores = (scores + b2_ref[...]) / temp_ref[0, 0]

    lmax = jnp.max(scores, axis=-1, keepdims=True)
    ex = jnp.exp(scores - lmax)
    w = ex / jnp.sum(ex, axis=-1, keepdims=True)
    weights_ref[...] = w

    iota = jax.lax.broadcasted_iota(jnp.int32, w.shape, 1)
    rem = w
    sel = jnp.zeros(w.shape, jnp.bool_)
    vals, idxs = [], []
    for _ in range(K):
        mk = jnp.max(rem, axis=-1, keepdims=True)
        ik = jnp.min(jnp.where(rem == mk, iota, e), axis=-1, keepdims=True)
        hit = iota == ik
        sel = jnp.logical_or(sel, hit)
        rem = jnp.where(hit, -1.0, rem)
        vals.append(mk)
        idxs.append(ik)
    sum_k = vals[0]
    for v in vals[1:]:
        sum_k = sum_k + v
    mask = jnp.where(sel, w / sum_k, 0.0)
    mask_ref[...] = mask
    idx_ref[...] = jnp.concatenate(idxs, axis=-1)

    @pl.when(m == 0)
    def _():
        usage_ref[...] = jnp.zeros_like(usage_ref)

    usage_ref[...] += jnp.sum(mask, axis=0, keepdims=True)

    @pl.when(m == nm - 1)
    def _():
        usage = usage_ref[...]
        ideal = jnp.sum(usage) / e
        lb = jnp.mean((usage - ideal) ** 2)
        cl = jnp.mean(jnp.maximum(usage - capacity, 0.0))
        loss_ref[...] = jnp.full((1, 1), lb + cl, jnp.float32)


def kernel(x, W1, b1, W2, b2, temperature):
    B, S, D = x.shape
    Dh, E = W2.shape
    M = B * S
    xf = x.reshape(M, D)
    b1r = b1.reshape(1, Dh)
    b2r = b2.reshape(1, E)
    tr = temperature.reshape(1, 1)
    capacity = float(int(1.25 * S))

    nm = M // M_TILE

    body = functools.partial(_router_body, e=E, capacity=capacity)
    mask, loss, weights, idx = pl.pallas_call(
        body,
        grid=(nm,),
        in_specs=[
            pl.BlockSpec((M_TILE, D), lambda m: (m, 0)),
            pl.BlockSpec((D, Dh), lambda m: (0, 0)),
            pl.BlockSpec((1, Dh), lambda m: (0, 0)),
            pl.BlockSpec((Dh, E), lambda m: (0, 0)),
            pl.BlockSpec((1, E), lambda m: (0, 0)),
            pl.BlockSpec((1, 1), lambda m: (0, 0)),
        ],
        out_specs=[
            pl.BlockSpec((M_TILE, E), lambda m: (m, 0)),
            pl.BlockSpec((1, 1), lambda m: (0, 0)),
            pl.BlockSpec((M_TILE, E), lambda m: (m, 0)),
            pl.BlockSpec((M_TILE, K), lambda m: (m, 0)),
        ],
        out_shape=[
            jax.ShapeDtypeStruct((M, E), jnp.float32),
            jax.ShapeDtypeStruct((1, 1), jnp.float32),
            jax.ShapeDtypeStruct((M, E), jnp.float32),
            jax.ShapeDtypeStruct((M, K), jnp.int32),
        ],
        scratch_shapes=[
            pltpu.VMEM((1, E), jnp.float32),
        ],
        compiler_params=pltpu.CompilerParams(
            dimension_semantics=("arbitrary",),
        ),
    )(xf, W1, b1r, W2, b2r, tr)

    return (mask.reshape(B, S, E), loss.reshape(()),
            weights.reshape(B, S, E), idx.reshape(B, S, K))
